# baseline (speedup 1.0000x reference)
import jax
import jax.numpy as jnp
from jax import lax
from jax.experimental import pallas as pl
from jax.experimental.pallas import tpu as pltpu
from jax.experimental.pallas import tpu_sc as plsc

_N = 8192
_E = 262144
_NC = 2
_NS = 16
_L = 16
_NW = _NC * _NS
_CHUNK = _E // _NW
_B = 64
_W = 32
_RING = 4
_MAXS = _B + _CHUNK // _W
_CAP = _CHUNK + _B * 8 + _W


def _gather_body(edges_hbm, score_hbm, out_hbm,
                 edges_v, hist_v, sorted_r, sorted_pos, out_v, bufs,
                 p_s, ends, slot_tc, slot_off, sem):
    wid = lax.axis_index("s") * _NC + lax.axis_index("c")
    base = pl.multiple_of(wid * _CHUNK, _CHUNK)
    lane = lax.iota(jnp.int32, _L)

    pltpu.sync_copy(edges_hbm.at[pl.ds(base * 2, _CHUNK * 2)], edges_v)

    def zero_hist(j, carry):
        hist_v[pl.ds(j * _L, _L)] = jnp.zeros((_L,), jnp.int32)
        return carry
    lax.fori_loop(0, (_L * _B) // _L, zero_hist, 0)

    def histo(k, carry):
        pos2 = (k * _L + lane) * 2
        c = plsc.load_gather(edges_v, [pos2 + 1])
        key = lane * _B + (c >> 7)
        old = plsc.load_gather(hist_v, [key])
        plsc.store_scatter(hist_v, [key], old + 1)
        return carry
    lax.fori_loop(0, _CHUNK // _L, histo, 0)

    def starts(b, a):
        col = plsc.load_gather(hist_v, [lane * _B + b])
        excl = plsc.cumsum(col) - col
        plsc.store_scatter(hist_v, [lane * _B + b], a + excl)
        p_s[b] = a
        nb = jnp.sum(col)
        ends[b] = a + nb
        return a + (((nb + 7) >> 3) << 3)
    lax.fori_loop(0, _B, starts, jnp.int32(0))

    def slots(b, ns):
        nb = ends[b] - p_s[b]
        wb = (nb + (_W - 1)) >> 5
        def one(w, carry):
            slot_tc[ns + w] = b
            slot_off[ns + w] = p_s[b] + w * _W
            return carry
        lax.fori_loop(0, wb, one, 0)
        return ns + wb
    n_slots = lax.fori_loop(0, _B, slots, jnp.int32(0))

    def pad(j, carry):
        sorted_r[pl.ds(j * _L, _L)] = jnp.zeros((_L,), jnp.int32)
        sorted_pos[pl.ds(j * _L, _L)] = jnp.zeros((_L,), jnp.int32)
        return carry
    lax.fori_loop(0, _CAP // _L, pad, 0)

    def permute(k, carry):
        pos = k * _L + lane
        r = plsc.load_gather(edges_v, [pos * 2])
        c = plsc.load_gather(edges_v, [pos * 2 + 1])
        key = lane * _B + (c >> 7)
        slot = plsc.load_gather(hist_v, [key])
        plsc.store_scatter(hist_v, [key], slot + 1)
        plsc.store_scatter(sorted_r, [slot], r)
        plsc.store_scatter(sorted_pos, [slot], pos)
        return carry
    lax.fori_loop(0, _CHUNK // _L, permute, 0)

    def issue(s):
        tc = slot_tc[s]
        off = pl.multiple_of(slot_off[s], 8)
        pltpu.async_copy(
            score_hbm.at[sorted_r.at[pl.ds(off, _W)], pl.ds(tc * 128, 128)],
            bufs.at[lax.rem(s, _RING)],
            sem,
        )

    def extract(s):
        jmod = lax.rem(s, _RING)
        tc = slot_tc[s]
        off = slot_off[s]
        nxt = ends[tc]
        def half(h, carry):
            q = off + h * _L + lane
            pos = plsc.load_gather(sorted_pos, [q])
            c = plsc.load_gather(edges_v, [pos * 2 + 1])
            vals = plsc.load_gather(
                bufs, [jnp.full((_L,), jmod, jnp.int32), h * _L + lane, c & 127]
            )
            plsc.store_scatter(out_v, [pos], vals, mask=q < nxt)
            return carry
        lax.fori_loop(0, _W // _L, half, 0)

    def drain(s):
        pltpu.make_async_copy(
            score_hbm.at[pl.ds(0, _W), pl.ds(0, 128)],
            bufs.at[lax.rem(s, _RING)],
            sem,
        ).wait()

    def main(s, carry):
        @pl.when(s >= _RING)
        def _():
            drain(s - _RING)
            extract(s - _RING)
        issue(s)
        return carry
    lax.fori_loop(0, n_slots, main, 0)

    def tail(s, carry):
        @pl.when(s >= 0)
        def _():
            drain(s)
            extract(s)
        return carry
    lax.fori_loop(jnp.maximum(n_slots - _RING, 0), n_slots, tail, 0)

    pltpu.sync_copy(out_v, out_hbm.at[pl.ds(base, _CHUNK)])


def kernel(inputs, edges, score_all):
    del inputs
    edges_flat = edges.astype(jnp.int32).reshape(-1)
    mesh = plsc.VectorSubcoreMesh(
        core_axis_name="c", subcore_axis_name="s",
        num_cores=_NC, num_subcores=_NS,
    )
    run = pl.kernel(
        _gather_body,
        out_type=jax.ShapeDtypeStruct((_E,), jnp.float32),
        mesh=mesh,
        compiler_params=pltpu.CompilerParams(needs_layout_passes=False),
        scratch_types=[
            pltpu.VMEM((_CHUNK * 2,), jnp.int32),
            pltpu.VMEM((_L * _B,), jnp.int32),
            pltpu.VMEM((_CAP,), jnp.int32),
            pltpu.VMEM((_CAP,), jnp.int32),
            pltpu.VMEM((_CHUNK,), jnp.float32),
            pltpu.VMEM((_RING, _W, 128), jnp.float32),
            pltpu.SMEM((_B,), jnp.int32),
            pltpu.SMEM((_B,), jnp.int32),
            pltpu.SMEM((_MAXS,), jnp.int32),
            pltpu.SMEM((_MAXS,), jnp.int32),
            pltpu.SemaphoreType.DMA,
        ],
    )
    return run(edges_flat, score_all)


# speedup vs baseline: 8.8930x; 8.8930x over previous
import jax
import jax.numpy as jnp
from jax import lax
from jax.experimental import pallas as pl
from jax.experimental.pallas import tpu as pltpu
from jax.experimental.pallas import tpu_sc as plsc

_N = 8192
_E = 262144
_NC = 2
_NS = 16
_L = 16
_NW = _NC * _NS
_CHUNK = _E // _NW


def _gather_body(e0_hbm, e1_hbm, score_hbm, out_hbm, e0_v, e1_v, out_v, sem):
    wid = lax.axis_index("s") * _NC + lax.axis_index("c")
    base = pl.multiple_of(wid * _CHUNK, _CHUNK)
    pltpu.sync_copy(e0_hbm.at[pl.ds(base, _CHUNK)], e0_v)
    pltpu.sync_copy(e1_hbm.at[pl.ds(base, _CHUNK)], e1_v)

    def step(k, carry):
        b = k * _L
        r = e0_v[pl.ds(b, _L)]
        c = e1_v[pl.ds(b, _L)]
        idx = ((r >> 3) << 16) | ((c >> 7) << 10) | ((r & 7) << 7) | (c & 127)
        pltpu.async_copy(score_hbm.at[idx], out_v.at[pl.ds(b, _L)], sem)
        return carry

    lax.fori_loop(0, _CHUNK // _L, step, 0)
    pltpu.make_async_copy(score_hbm.at[pl.ds(0, _CHUNK)], out_v, sem).wait()
    pltpu.sync_copy(out_v, out_hbm.at[pl.ds(base, _CHUNK)])


def kernel(inputs, edges, score_all):
    del inputs
    edges = edges.astype(jnp.int32)
    e0 = edges[:, 0]
    e1 = edges[:, 1]
    score_phys = (
        score_all.reshape(1024, 8, 64, 128)
        .transpose(0, 2, 1, 3)
        .reshape(_N * _N)
    )
    mesh = plsc.VectorSubcoreMesh(
        core_axis_name="c", subcore_axis_name="s",
        num_cores=_NC, num_subcores=_NS,
    )
    run = pl.kernel(
        _gather_body,
        out_type=jax.ShapeDtypeStruct((_E,), jnp.float32),
        mesh=mesh,
        compiler_params=pltpu.CompilerParams(needs_layout_passes=False),
        scratch_types=[
            pltpu.VMEM((_CHUNK,), jnp.int32),
            pltpu.VMEM((_CHUNK,), jnp.int32),
            pltpu.VMEM((_CHUNK,), jnp.float32),
            pltpu.SemaphoreType.DMA,
        ],
    )
    return run(e0, e1, score_phys)


# speedup vs baseline: 9.0391x; 1.0164x over previous
import jax
import jax.numpy as jnp
from jax import lax
from jax.experimental import pallas as pl
from jax.experimental.pallas import tpu as pltpu
from jax.experimental.pallas import tpu_sc as plsc

_N = 8192
_E = 262144
_NC = 2
_NS = 16
_L = 16
_NW = _NC * _NS
_CHUNK = _E // _NW


def _gather_body(e0_hbm, e1_hbm, score_hbm, out_hbm, e0_v, e1_v, out_v, sem):
    wid = lax.axis_index("s") * _NC + lax.axis_index("c")
    base = pl.multiple_of(wid * _CHUNK, _CHUNK)
    cp0 = pltpu.async_copy(e0_hbm.at[pl.ds(base, _CHUNK)], e0_v, sem)
    cp1 = pltpu.async_copy(e1_hbm.at[pl.ds(base, _CHUNK)], e1_v, sem)
    cp0.wait()
    cp1.wait()

    def step(k, carry):
        b = k * _L
        r = e0_v[pl.ds(b, _L)]
        c = e1_v[pl.ds(b, _L)]
        idx = ((r >> 3) << 16) | ((c >> 7) << 10) | ((r & 7) << 7) | (c & 127)
        pltpu.async_copy(score_hbm.at[idx], out_v.at[pl.ds(b, _L)], sem)
        return carry

    lax.fori_loop(0, _CHUNK // _L, step, 0)
    pltpu.make_async_copy(score_hbm.at[pl.ds(0, _CHUNK)], out_v, sem).wait()
    pltpu.sync_copy(out_v, out_hbm.at[pl.ds(base, _CHUNK)])


def kernel(inputs, edges, score_all):
    del inputs
    edges = edges.astype(jnp.int32)
    e0 = edges[:, 0]
    e1 = edges[:, 1]
    score_phys = (
        score_all.reshape(1024, 8, 64, 128)
        .transpose(0, 2, 1, 3)
        .reshape(_N * _N)
    )
    mesh = plsc.VectorSubcoreMesh(
        core_axis_name="c", subcore_axis_name="s",
        num_cores=_NC, num_subcores=_NS,
    )
    run = pl.kernel(
        _gather_body,
        out_type=jax.ShapeDtypeStruct((_E,), jnp.float32),
        mesh=mesh,
        compiler_params=pltpu.CompilerParams(
            needs_layout_passes=False,
            disable_bounds_checks=True,
            disable_semaphore_checks=True,
        ),
        scratch_types=[
            pltpu.VMEM((_CHUNK,), jnp.int32),
            pltpu.VMEM((_CHUNK,), jnp.int32),
            pltpu.VMEM((_CHUNK,), jnp.float32),
            pltpu.SemaphoreType.DMA,
        ],
    )
    return run(e0, e1, score_phys)
